# Initial kernel scaffold; baseline (speedup 1.0000x reference)
import jax
import jax.numpy as jnp
from jax.experimental import pallas as pl


def kernel(gate_type, edge_index, edge_attr, embed, W1, b1, W2, b2, gamma, beta):
    raise NotImplementedError("write your pallas kernel here")


# speedup vs baseline: 3.9530x; 3.9530x over previous
import functools

import jax
import jax.numpy as jnp
from jax import lax
from jax.experimental import pallas as pl
from jax.experimental.pallas import tpu as pltpu
from jax.experimental.pallas import tpu_sc as plsc

N = 10000
E = 320000
D = 128
DE = 16
NUM_GATE_TYPES = 32
NUM_LAYERS = 3
BN_EPS = 1e-5

NC = 2
NS = 16
NW = NC * NS
NPAD = 10240
ROWS_PER_SUB = NPAD // NS
CHUNK = 128
NCHUNKS = 79
EPW = CHUNK * NCHUNKS
EP = EPW * NW


@functools.lru_cache(maxsize=None)
def _make_sc_seg_sum(gather):
    mesh = plsc.VectorSubcoreMesh(core_axis_name="c", subcore_axis_name="s")

    @functools.partial(
        pl.kernel,
        out_type=jax.ShapeDtypeStruct((NC, NPAD, D), jnp.float32),
        mesh=mesh,
        scratch_types=[
            pltpu.VMEM((CHUNK,), jnp.int32),
            pltpu.VMEM((CHUNK,), jnp.int32),
            pltpu.VMEM((CHUNK, D), jnp.float32),
            pltpu.VMEM_SHARED((NPAD, D), jnp.float32),
            pltpu.SemaphoreType.DMA,
        ],
    )
    def k(table_hbm, src_hbm, dst_hbm, zeros_hbm, out_hbm,
          src_v, dst_v, rows_v, acc_sh, sem):
        c = lax.axis_index("c")
        s = lax.axis_index("s")
        wid = s * NC + c
        r0 = s * ROWS_PER_SUB
        pltpu.sync_copy(zeros_hbm.at[pl.ds(r0, ROWS_PER_SUB)],
                        acc_sh.at[pl.ds(r0, ROWS_PER_SUB)])
        plsc.subcore_barrier()

        base_w = wid * EPW

        def body(i, _):
            base = base_w + i * CHUNK
            pltpu.sync_copy(dst_hbm.at[pl.ds(base, CHUNK)], dst_v)
            if gather:
                pltpu.sync_copy(src_hbm.at[pl.ds(base, CHUNK)], src_v)
                pltpu.async_copy(table_hbm.at[src_v], rows_v, sem).wait()
            else:
                pltpu.sync_copy(table_hbm.at[pl.ds(base, CHUNK)], rows_v)
            pltpu.sync_copy(rows_v, acc_sh.at[dst_v], add=True)
            return 0

        lax.fori_loop(0, NCHUNKS, body, 0)
        plsc.subcore_barrier()
        pltpu.sync_copy(acc_sh.at[pl.ds(r0, ROWS_PER_SUB)],
                        out_hbm.at[c, pl.ds(r0, ROWS_PER_SUB)])

    return k


def _embed_body(gt_ref, embed_ref, out_ref):
    gt = gt_ref[...]
    ids = lax.broadcasted_iota(jnp.int32, (1, NUM_GATE_TYPES), 1)
    onehot = jnp.where(gt == ids, 1.0, 0.0).astype(jnp.float32)
    out_ref[...] = jnp.dot(onehot, embed_ref[...],
                           preferred_element_type=jnp.float32, precision=lax.Precision.HIGHEST)


def _tc_embed(gate_type, embed):
    return pl.pallas_call(
        _embed_body,
        out_shape=jax.ShapeDtypeStruct((N, D), jnp.float32),
    )(gate_type.reshape(N, 1), embed)


def _bdot(a, b):
    return jnp.dot(a.astype(jnp.bfloat16), b.astype(jnp.bfloat16),
                   preferred_element_type=jnp.float32)


def _dense_body(h_ref, ph_ref, pw_ref, w1a_ref, w1b_ref, b1_ref,
                w2_ref, b2_ref, g_ref, bt_ref, out_ref):
    t = h_ref[...] + ph_ref[0, :N, :] + ph_ref[1, :N, :]
    u = pw_ref[0, :N, :DE] + pw_ref[1, :N, :DE]
    x = (_bdot(t, w1a_ref[...]) + _bdot(u, w1b_ref[...]) + b1_ref[...])
    x = jnp.maximum(x, 0.0)
    x = _bdot(x, w2_ref[...]) + b2_ref[...]
    mu = jnp.mean(x, axis=0, keepdims=True)
    var = jnp.mean((x - mu) ** 2, axis=0, keepdims=True)
    xn = (x - mu) * lax.rsqrt(var + BN_EPS) * g_ref[...] + bt_ref[...]
    out_ref[...] = jnp.where(xn >= 0, xn, 0.01 * xn)


def _tc_dense(h, ph, pw, w1, b1, w2, b2, gamma, beta):
    w1a = w1[:D, :]
    w1b = w1[D:, :]
    return pl.pallas_call(
        _dense_body,
        out_shape=jax.ShapeDtypeStruct((N, D), jnp.float32),
    )(h, ph, pw, w1a, w1b, b1.reshape(1, D), w2, b2.reshape(1, D),
      gamma.reshape(1, D), beta.reshape(1, D))


def kernel(gate_type, edge_index, edge_attr, embed, W1, b1, W2, b2, gamma, beta):
    src = jnp.concatenate(
        [edge_index[0], jnp.zeros((EP - E,), edge_index.dtype)])
    dst = jnp.concatenate(
        [edge_index[1], jnp.full((EP - E,), NPAD - 1, edge_index.dtype)])
    ea_pad = jnp.zeros((EP, D), jnp.float32).at[:E, :DE].set(edge_attr)
    zeros = jnp.zeros((NPAD, D), jnp.float32)

    h = _tc_embed(gate_type, embed)
    pw = _make_sc_seg_sum(False)(ea_pad, src, dst, zeros)
    for l in range(NUM_LAYERS):
        ph = _make_sc_seg_sum(True)(h, src, dst, zeros)
        h = _tc_dense(h, ph, pw, W1[l], b1[l], W2[l], b2[l], gamma[l], beta[l])
    return h
